# Initial kernel scaffold; baseline (speedup 1.0000x reference)
import jax
import jax.numpy as jnp
from jax.experimental import pallas as pl


def kernel(preds, targets, img_size):
    raise NotImplementedError("write your pallas kernel here")


# speedup vs baseline: 115.0001x; 115.0001x over previous
import functools

import jax
import jax.numpy as jnp
from jax.experimental import pallas as pl

_B = 3
_C = 20
_T = 20
_GS = 52
_CELLS = _GS * _GS
_NSC = 2
_BATCH = 16
_IGNORE = 0.7
_AT = ((10.0, 13.0), (16.0, 30.0), (33.0, 23.0),
       (30.0, 61.0), (62.0, 45.0), (59.0, 119.0))


def _cum_and(v):
    sh = 1
    while sh < _T:
        shifted = jnp.pad(v, ((sh, 0), (0, 0)), constant_values=1.0)[:_T, :]
        v = v * shifted
        sh *= 2
    return v


def _sig(x):
    return 1.0 / (1.0 + jnp.exp(-x))


def _yolo_kernel(pred_ref, tgt_ref, img_ref, out_ref):
    idx = pl.program_id(0)
    p = pred_ref[0, 0]
    tg = tgt_ref[0]
    img = img_ref[0, 0:1]
    img = img.reshape(1, 1)
    gs_f = float(_GS)

    tx = tg[:, 1:2]
    ty = tg[:, 2:3]
    tw = tg[:, 3:4]
    th = tg[:, 4:5]
    valid = _cum_and((tx != 0.0).astype(jnp.float32)) > 0.5

    a2 = tw * th
    b2x1 = 0.0 - tw / 2.0
    b2x2 = 0.0 + tw / 2.0
    b2y1 = 0.0 - th / 2.0
    b2y2 = 0.0 + th / 2.0
    best_iou = None
    best_kf = None
    for k in range(2 * _B):
        awp = _AT[k][0] / img
        ahp = _AT[k][1] / img
        b1x1 = 0.0 - awp / 2.0
        b1x2 = 0.0 + awp / 2.0
        b1y1 = 0.0 - ahp / 2.0
        b1y2 = 0.0 + ahp / 2.0
        iw = jnp.maximum(jnp.minimum(b1x2, b2x2) - jnp.maximum(b1x1, b2x1), 0.0)
        ih = jnp.maximum(jnp.minimum(b1y2, b2y2) - jnp.maximum(b1y1, b2y1), 0.0)
        inter = iw * ih
        a1 = awp * ahp
        iou = inter / (a1 + a2 - inter + 1e-16)
        if best_iou is None:
            best_iou, best_kf = iou, jnp.zeros_like(iou)
        else:
            upd = iou > best_iou
            best_iou = jnp.where(upd, iou, best_iou)
            best_kf = jnp.where(upd, float(k), best_kf)

    on_s1 = best_kf >= float(_B)
    in_scale = on_s1 == (idx == 1)
    a_t = best_kf - jnp.where(on_s1, float(_B), 0.0)
    apply_t = valid & in_scale

    aw_sel = jnp.zeros_like(best_kf)
    ah_sel = jnp.zeros_like(best_kf)
    for k in range(2 * _B):
        hit = best_kf == float(k)
        aw_sel = jnp.where(hit, _AT[k][0], aw_sel)
        ah_sel = jnp.where(hit, _AT[k][1], ah_sel)

    i_f = jnp.floor(tx * gs_f)
    j_f = jnp.floor(ty * gs_f)
    enc0 = tx * gs_f - i_f
    enc1 = ty * gs_f - j_f
    enc2 = jnp.log(tw * img / aw_sel)
    enc3 = jnp.log(th * img / ah_sel)
    s_t = 2.0 - tw * th
    cid_t = j_f * gs_f + i_f

    twe = jnp.where(valid, tw, 0.0)
    the = jnp.where(valid, th, 0.0)
    tx1 = tx - twe / 2.0
    tx2 = tx + twe / 2.0
    ty1 = ty - the / 2.0
    ty2 = ty + the / 2.0
    c_t = twe * the + 1e-16

    it = jax.lax.broadcasted_iota(jnp.int32, (1, _CELLS), 1)
    gx = (it % _GS).astype(jnp.float32)
    gy = (it // _GS).astype(jnp.float32)
    c_f = it.astype(jnp.float32)
    t_iota = jax.lax.broadcasted_iota(jnp.int32, (_T, 1), 0).astype(jnp.float32)
    cls_is0 = jax.lax.broadcasted_iota(jnp.int32, (_C, 1), 0) == 0

    acc = jnp.zeros((), jnp.float32)
    for a in range(_B):
        base = a * (_C + 5)
        sx = _sig(p[base + 0:base + 1, :])
        sy = _sig(p[base + 1:base + 2, :])
        rw = p[base + 2:base + 3, :]
        rh = p[base + 3:base + 4, :]
        cf = _sig(p[base + 4:base + 5, :])
        pc = _sig(p[base + 5:base + _C + 5, :])

        aw_s = jnp.where(idx == 0, _AT[a][0], _AT[_B + a][0])
        ah_s = jnp.where(idx == 0, _AT[a][1], _AT[_B + a][1])
        cx = (sx + gx) / gs_f
        cy = (sy + gy) / gs_f
        cw = jnp.exp(rw) * aw_s / img
        ch = jnp.exp(rh) * ah_s / img
        px1 = cx - cw / 2.0
        px2 = cx + cw / 2.0
        py1 = cy - ch / 2.0
        py2 = cy + ch / 2.0
        area1 = cw * ch

        iw = jnp.maximum(jnp.minimum(px2, tx2) - jnp.maximum(px1, tx1), 0.0)
        ih = jnp.maximum(jnp.minimum(py2, ty2) - jnp.maximum(py1, ty1), 0.0)
        inter = iw * ih
        cond = inter * (1.0 + _IGNORE) <= (area1 + c_t) * _IGNORE
        noobj = jnp.all(cond, axis=0, keepdims=True)

        match = apply_t & (a_t == float(a)) & (cid_t == c_f)
        owner = jnp.max(jnp.where(match, t_iota, -1.0), axis=0, keepdims=True)
        obj = owner >= 0.0
        sel = t_iota == owner

        def pick(v):
            return jnp.sum(jnp.where(sel, v, 0.0), axis=0, keepdims=True)

        e0 = pick(enc0)
        e1 = pick(enc1)
        e2 = pick(enc2)
        e3 = pick(enc3)
        sc = pick(s_t)

        l_obj = jnp.sum(jnp.where(obj, cf - 1.0, 0.0) ** 2)
        l_noobj = jnp.sum(jnp.where(noobj & (~obj), cf, 0.0) ** 2)
        tgt_cls = jnp.where(cls_is0 & obj, 1.0, 0.0)
        l_cls = jnp.sum(jnp.where(obj, pc - tgt_cls, 0.0) ** 2)
        d0 = sx * sc - e0 * sc
        d1 = sy * sc - e1 * sc
        d2 = rw * sc - e2 * sc
        d3 = rh * sc - e3 * sc
        l_coord = jnp.sum(jnp.where(obj, d0, 0.0) ** 2)
        l_coord += jnp.sum(jnp.where(obj, d1, 0.0) ** 2)
        l_coord += jnp.sum(jnp.where(obj, d2, 0.0) ** 2)
        l_coord += jnp.sum(jnp.where(obj, d3, 0.0) ** 2)
        acc = acc + (l_obj + l_noobj + l_cls + l_coord)

    out_ref[0, 0, :, :] = (acc / float(_BATCH)).reshape(1, 1)


@functools.partial(jax.jit, static_argnames=())
def kernel(preds, targets, img_size):
    pr = preds.reshape(_NSC, _BATCH, _B * (_C + 5), _CELLS)
    img = jnp.asarray(img_size, jnp.float32).reshape(1, 1)
    partials = pl.pallas_call(
        _yolo_kernel,
        grid=(_NSC, _BATCH),
        in_specs=[
            pl.BlockSpec((1, 1, _B * (_C + 5), _CELLS), lambda s, b: (s, b, 0, 0)),
            pl.BlockSpec((1, _T, 5), lambda s, b: (b, 0, 0)),
            pl.BlockSpec((1, 1), lambda s, b: (0, 0)),
        ],
        out_specs=pl.BlockSpec((1, 1, 1, 1), lambda s, b: (s, b, 0, 0)),
        out_shape=jax.ShapeDtypeStruct((_NSC, _BATCH, 1, 1), jnp.float32),
    )(pr, targets, img)
    return jnp.sum(partials)
